# baseline (speedup 1.0000x reference)
import jax
import jax.numpy as jnp
from jax.experimental import pallas as pl
from jax.experimental.pallas import tpu as pltpu

B, N, D = 32, 576, 768
N_CLASSES, K = 200, 5
C = N_CLASSES + 1
CK = C * K
CKP = 1024
TEMPERATURE = 0.2


def _proto_prep_kernel(pT_ref, out_ref):
    p = pT_ref[...]
    col = jax.lax.broadcasted_iota(jnp.int32, (1, CKP), 1)
    valid = col < CK
    ssq = jnp.sum(p * p, axis=0, keepdims=True)
    inv = jax.lax.rsqrt(jnp.maximum(ssq, 1e-24))
    pn = jnp.where(valid, p * inv, 0.0)
    out_ref[...] = pn.astype(jnp.bfloat16)


def _main_kernel(x_ref, pT_ref, sa_ref, out_ref, img_ref, cls_ref):
    x = x_ref[...]
    ssq = jnp.sum(x * x, axis=1, keepdims=True)
    xn = (x * jax.lax.rsqrt(jnp.maximum(ssq, 1e-24))).astype(jnp.bfloat16)
    acc = jax.lax.dot_general(
        xn, pT_ref[...], (((1,), (0,)), ((), ())),
        preferred_element_type=jnp.float32)
    out_ref[...] = acc
    maxv = jnp.max(acc, axis=0, keepdims=True)
    img_ref[...] = maxv[:, :CK].reshape(1, 1, CK)

    j_col = jax.lax.broadcasted_iota(jnp.int32, (1, CKP), 1)
    jg = jax.lax.broadcasted_iota(jnp.int32, (CKP, N_CLASSES), 0) // K
    cg = jax.lax.broadcasted_iota(jnp.int32, (CKP, N_CLASSES), 1)
    G = (jg == cg).astype(jnp.float32)
    e = jnp.exp(sa_ref[...])
    S = jax.lax.dot_general(e, G, (((1,), (0,)), ((), ())),
                            preferred_element_type=jnp.float32)
    jg2 = jax.lax.broadcasted_iota(jnp.int32, (N_CLASSES, CKP), 1) // K
    cg2 = jax.lax.broadcasted_iota(jnp.int32, (N_CLASSES, CKP), 0)
    G2 = (jg2 == cg2).astype(jnp.float32)
    denom = jax.lax.dot_general(S, G2, (((1,), (0,)), ((), ())),
                                preferred_element_type=jnp.float32)
    valid = j_col < N_CLASSES * K
    sa_col = jnp.where(valid, e * K / jnp.maximum(denom, 1e-30), 0.0)
    w = maxv * sa_col
    cls = jax.lax.dot_general(w, G, (((1,), (0,)), ((), ())),
                              preferred_element_type=jnp.float32)
    cls_ref[...] = (cls * (1.0 / TEMPERATURE)).reshape(1, 1, N_CLASSES)


def kernel(x, prototypes, sa_weights):
    x2d = x.reshape(B * N, D)
    pT = prototypes.reshape(CK, D).T
    sa_flat = jnp.pad(sa_weights.reshape(1, N_CLASSES * K),
                      ((0, 0), (0, CKP - N_CLASSES * K)))

    pnT = pl.pallas_call(
        _proto_prep_kernel,
        grid=(1,),
        out_shape=jax.ShapeDtypeStruct((D, CKP), jnp.bfloat16),
        in_specs=[pl.BlockSpec((D, CKP), lambda i: (0, 0))],
        out_specs=pl.BlockSpec((D, CKP), lambda i: (0, 0)),
    )(pT)

    logits2d, img2d, cls = pl.pallas_call(
        _main_kernel,
        grid=(B,),
        out_shape=(
            jax.ShapeDtypeStruct((B * N, CK), jnp.float32),
            jax.ShapeDtypeStruct((B, 1, CK), jnp.float32),
            jax.ShapeDtypeStruct((B, 1, N_CLASSES), jnp.float32),
        ),
        in_specs=[
            pl.BlockSpec((N, D), lambda i: (i, 0)),
            pl.BlockSpec((D, CKP), lambda i: (0, 0)),
            pl.BlockSpec((1, CKP), lambda i: (0, 0)),
        ],
        out_specs=(
            pl.BlockSpec((N, CKP), lambda i: (i, 0)),
            pl.BlockSpec((1, 1, CK), lambda i: (i, 0, 0)),
            pl.BlockSpec((1, 1, N_CLASSES), lambda i: (i, 0, 0)),
        ),
        compiler_params=pltpu.CompilerParams(
            dimension_semantics=("arbitrary",)),
    )(x2d, pnT, sa_flat)

    patch_prototype_logits = logits2d.reshape(B, N, C, K)
    image_prototype_logits = img2d.reshape(B, C, K)
    return (patch_prototype_logits, image_prototype_logits,
            cls.reshape(B, N_CLASSES))
